# Initial kernel scaffold; baseline (speedup 1.0000x reference)
import jax
import jax.numpy as jnp
from jax.experimental import pallas as pl


def kernel(user_song_adj, song_artist_adj, user_emb, song_emb, artist_emb, W_as, b_as, W_s, b_s, W_su, b_su, W_u, b_u):
    raise NotImplementedError("write your pallas kernel here")


# speedup vs baseline: 1.1334x; 1.1334x over previous
import functools

import jax
import jax.numpy as jnp
from jax.experimental import pallas as pl


_BLK = 1000


def _update_block(h_ref, aggr_ref, w1t_ref, b1_ref, w2at_ref, w2bt_ref, b2_ref, out_ref):
    aggr = aggr_ref[...]
    msg = jnp.maximum(
        jnp.dot(aggr, w1t_ref[...], preferred_element_type=jnp.float32) + b1_ref[...],
        0.0,
    )
    out = (
        jnp.dot(h_ref[...], w2at_ref[...], preferred_element_type=jnp.float32)
        + jnp.dot(msg, w2bt_ref[...], preferred_element_type=jnp.float32)
        + b2_ref[...]
    )
    n = jnp.sqrt(jnp.sum(out * out, axis=1, keepdims=True))
    out_ref[...] = out / jnp.maximum(n, 1e-12)


def _dense_update(h, aggr, W1, b1, W2, b2):
    n, hdim = h.shape
    assert n % _BLK == 0
    grid = (n // _BLK,)
    w1t = W1.T
    w2at = W2[:, :hdim].T
    w2bt = W2[:, hdim:].T
    b1r = b1.reshape(1, hdim)
    b2r = b2.reshape(1, hdim)
    row_spec = pl.BlockSpec((_BLK, hdim), lambda i: (i, 0))
    full_spec = pl.BlockSpec((hdim, hdim), lambda i: (0, 0))
    bias_spec = pl.BlockSpec((1, hdim), lambda i: (0, 0))
    return pl.pallas_call(
        _update_block,
        grid=grid,
        in_specs=[row_spec, row_spec, full_spec, bias_spec, full_spec, full_spec, bias_spec],
        out_specs=row_spec,
        out_shape=jax.ShapeDtypeStruct((n, hdim), jnp.float32),
    )(h, aggr, w1t, b1r, w2at, w2bt, b2r)


def _aggregate(dst_idx, src_idx, table, num_dst):
    edge_feats = jnp.take(table, src_idx, axis=0)
    return jnp.zeros((num_dst, table.shape[1]), jnp.float32).at[dst_idx].add(edge_feats)


def kernel(user_song_adj, song_artist_adj, user_emb, song_emb, artist_emb,
           W_as, b_as, W_s, b_s, W_su, b_su, W_u, b_u):
    num_users = user_emb.shape[0]
    num_songs = song_emb.shape[0]
    aggr_artist = _aggregate(song_artist_adj[0], song_artist_adj[1], artist_emb, num_songs)
    h_s_new = _dense_update(song_emb, aggr_artist, W_as, b_as, W_s, b_s)
    aggr_song = _aggregate(user_song_adj[0], user_song_adj[1], h_s_new, num_users)
    h_u_new = _dense_update(user_emb, aggr_song, W_su, b_su, W_u, b_u)
    return (h_u_new, h_s_new)


# speedup vs baseline: 5.3927x; 4.7581x over previous
import functools

import jax
import jax.numpy as jnp
from jax import lax
from jax.experimental import pallas as pl
from jax.experimental.pallas import tpu as pltpu
from jax.experimental.pallas import tpu_sc as plsc

_NC = 2
_NS = 16
_H = 64

_CROWS = 2
_CHUNK = _CROWS * 128


def _sc_aggregate_body(nrows_table, e_rows, half, acc_rows, zrows, chunks_per_tile,
                       src2d, dst2d, table, zeros, out,
                       idx_src, adj, rows, acc, sem):
    c = lax.axis_index("c")
    s = lax.axis_index("s")
    base = c * half

    pltpu.sync_copy(zeros, acc.at[pl.ds(s * zrows, zrows)])
    plsc.subcore_barrier()

    rows_per_tile = e_rows // _NS

    def chunk_body(k, carry):
        row0 = s * rows_per_tile + k * _CROWS
        pltpu.sync_copy(src2d.at[pl.ds(row0, _CROWS)], idx_src)
        pltpu.sync_copy(dst2d.at[pl.ds(row0, _CROWS)], adj)
        for j in range(_CROWS):
            for g in range(8):
                d = adj[j, pl.ds(g * 16, 16)]
                loc = d - base
                ok = (loc >= 0) & (loc < half)
                adj[j, pl.ds(g * 16, 16)] = jnp.where(ok, loc, half)
        cps = [
            pltpu.async_copy(table.at[idx_src.at[j]],
                             rows.at[pl.ds(j * 128, 128)], sem)
            for j in range(_CROWS)
        ]
        for cp in cps:
            cp.wait()
        for j in range(_CROWS):
            pltpu.sync_copy(rows.at[pl.ds(j * 128, 128)], acc.at[adj.at[j]],
                            add=True)
        return carry

    lax.fori_loop(0, chunks_per_tile, chunk_body, 0)
    plsc.subcore_barrier()

    big = (half + _NS - 1) // _NS
    big = ((big + 7) // 8) * 8
    last = half - (_NS - 1) * big

    @pl.when(s < _NS - 1)
    def _():
        pltpu.sync_copy(acc.at[pl.ds(s * big, big)],
                        out.at[pl.ds(base + s * big, big)])

    @pl.when(s == _NS - 1)
    def _():
        pltpu.sync_copy(acc.at[pl.ds((_NS - 1) * big, last)],
                        out.at[pl.ds(base + (_NS - 1) * big, last)])


def _sc_aggregate(dst_idx, src_idx, table, num_dst):
    e = dst_idx.shape[0]
    nrows_table = table.shape[0]
    assert num_dst % _NC == 0
    half = num_dst // _NC
    acc_rows = half + 8
    assert acc_rows % _NS == 0
    zrows = acc_rows // _NS

    e_pad = ((e + _NS * _CHUNK - 1) // (_NS * _CHUNK)) * (_NS * _CHUNK)
    pad = e_pad - e
    if pad:
        src_idx = jnp.concatenate([src_idx, jnp.zeros((pad,), jnp.int32)])
        dst_idx = jnp.concatenate(
            [dst_idx, jnp.full((pad,), num_dst, jnp.int32)])
    src2d = src_idx.reshape(-1, 128)
    dst2d = dst_idx.reshape(-1, 128)
    e_rows = e_pad // 128
    chunks_per_tile = e_pad // (_NS * _CHUNK)
    zeros = jnp.zeros((zrows, _H), jnp.float32)

    mesh = plsc.VectorSubcoreMesh(core_axis_name="c", subcore_axis_name="s",
                                  num_cores=_NC, num_subcores=_NS)
    body = functools.partial(_sc_aggregate_body, nrows_table, e_rows, half,
                             acc_rows, zrows, chunks_per_tile)
    fn = pl.kernel(
        body,
        out_type=jax.ShapeDtypeStruct((num_dst, _H), jnp.float32),
        mesh=mesh,
        scratch_types=[
            pltpu.VMEM((_CROWS, 128), jnp.int32),
            pltpu.VMEM((_CROWS, 128), jnp.int32),
            pltpu.VMEM((_CHUNK, _H), jnp.float32),
            pltpu.VMEM_SHARED((acc_rows, _H), jnp.float32),
            pltpu.SemaphoreType.DMA,
        ],
        compiler_params=pltpu.CompilerParams(use_tc_tiling_on_sc=False),
    )
    return fn(src2d, dst2d, table, zeros)


_BLK = 1000


def _update_block(h_ref, aggr_ref, w1t_ref, b1_ref, w2at_ref, w2bt_ref, b2_ref, out_ref):
    aggr = aggr_ref[...]
    msg = jnp.maximum(
        jnp.dot(aggr, w1t_ref[...], preferred_element_type=jnp.float32) + b1_ref[...],
        0.0,
    )
    out = (
        jnp.dot(h_ref[...], w2at_ref[...], preferred_element_type=jnp.float32)
        + jnp.dot(msg, w2bt_ref[...], preferred_element_type=jnp.float32)
        + b2_ref[...]
    )
    n = jnp.sqrt(jnp.sum(out * out, axis=1, keepdims=True))
    out_ref[...] = out / jnp.maximum(n, 1e-12)


def _dense_update(h, aggr, W1, b1, W2, b2):
    n, hdim = h.shape
    assert n % _BLK == 0
    grid = (n // _BLK,)
    w1t = W1.T
    w2at = W2[:, :hdim].T
    w2bt = W2[:, hdim:].T
    b1r = b1.reshape(1, hdim)
    b2r = b2.reshape(1, hdim)
    row_spec = pl.BlockSpec((_BLK, hdim), lambda i: (i, 0))
    full_spec = pl.BlockSpec((hdim, hdim), lambda i: (0, 0))
    bias_spec = pl.BlockSpec((1, hdim), lambda i: (0, 0))
    return pl.pallas_call(
        _update_block,
        grid=grid,
        in_specs=[row_spec, row_spec, full_spec, bias_spec, full_spec, full_spec, bias_spec],
        out_specs=row_spec,
        out_shape=jax.ShapeDtypeStruct((n, hdim), jnp.float32),
    )(h, aggr, w1t, b1r, w2at, w2bt, b2r)


def kernel(user_song_adj, song_artist_adj, user_emb, song_emb, artist_emb,
           W_as, b_as, W_s, b_s, W_su, b_su, W_u, b_u):
    num_users = user_emb.shape[0]
    num_songs = song_emb.shape[0]
    aggr_artist = _sc_aggregate(song_artist_adj[0], song_artist_adj[1],
                                artist_emb, num_songs)
    h_s_new = _dense_update(song_emb, aggr_artist, W_as, b_as, W_s, b_s)
    aggr_song = _sc_aggregate(user_song_adj[0], user_song_adj[1],
                              h_s_new, num_users)
    h_u_new = _dense_update(user_emb, aggr_song, W_su, b_su, W_u, b_u)
    return (h_u_new, h_s_new)


# speedup vs baseline: 5.4980x; 1.0195x over previous
import functools

import jax
import jax.numpy as jnp
from jax import lax
from jax.experimental import pallas as pl
from jax.experimental.pallas import tpu as pltpu
from jax.experimental.pallas import tpu_sc as plsc

_NC = 2
_NS = 16
_H = 64

_CROWS = 2
_CHUNK = _CROWS * 128


def _sc_aggregate_body(nrows_table, e_rows, half, acc_rows, zrows, chunks_per_tile,
                       src2d, dst2d, table, zeros, out,
                       idx_src, adj, rows, acc, sem):
    c = lax.axis_index("c")
    s = lax.axis_index("s")
    base = c * half

    pltpu.sync_copy(zeros, acc.at[pl.ds(s * zrows, zrows)])
    plsc.subcore_barrier()

    rows_per_tile = e_rows // _NS

    def chunk_body(k, carry):
        row0 = s * rows_per_tile + k * _CROWS
        pltpu.sync_copy(src2d.at[pl.ds(row0, _CROWS)], idx_src)
        pltpu.sync_copy(dst2d.at[pl.ds(row0, _CROWS)], adj)
        for j in range(_CROWS):
            for g in range(8):
                d = adj[j, pl.ds(g * 16, 16)]
                loc = d - base
                ok = (loc >= 0) & (loc < half)
                adj[j, pl.ds(g * 16, 16)] = jnp.where(ok, loc, half + (d & 15))
        cps = [
            pltpu.async_copy(table.at[idx_src.at[j]],
                             rows.at[pl.ds(j * 128, 128)], sem)
            for j in range(_CROWS)
        ]
        for cp in cps:
            cp.wait()
        for j in range(_CROWS):
            pltpu.sync_copy(rows.at[pl.ds(j * 128, 128)], acc.at[adj.at[j]],
                            add=True)
        return carry

    lax.fori_loop(0, chunks_per_tile, chunk_body, 0)
    plsc.subcore_barrier()

    big = (half + _NS - 1) // _NS
    big = ((big + 7) // 8) * 8
    last = half - (_NS - 1) * big

    @pl.when(s < _NS - 1)
    def _():
        pltpu.sync_copy(acc.at[pl.ds(s * big, big)],
                        out.at[pl.ds(base + s * big, big)])

    @pl.when(s == _NS - 1)
    def _():
        pltpu.sync_copy(acc.at[pl.ds((_NS - 1) * big, last)],
                        out.at[pl.ds(base + (_NS - 1) * big, last)])


def _sc_aggregate(dst_idx, src_idx, table, num_dst):
    e = dst_idx.shape[0]
    nrows_table = table.shape[0]
    assert num_dst % _NC == 0
    half = num_dst // _NC
    acc_rows = half + 24
    assert acc_rows % _NS == 0
    zrows = acc_rows // _NS

    e_pad = ((e + _NS * _CHUNK - 1) // (_NS * _CHUNK)) * (_NS * _CHUNK)
    pad = e_pad - e
    if pad:
        src_idx = jnp.concatenate([src_idx, jnp.zeros((pad,), jnp.int32)])
        dst_idx = jnp.concatenate(
            [dst_idx, jnp.full((pad,), num_dst, jnp.int32)])
    src2d = src_idx.reshape(-1, 128)
    dst2d = dst_idx.reshape(-1, 128)
    e_rows = e_pad // 128
    chunks_per_tile = e_pad // (_NS * _CHUNK)
    zeros = jnp.zeros((zrows, _H), jnp.float32)

    mesh = plsc.VectorSubcoreMesh(core_axis_name="c", subcore_axis_name="s",
                                  num_cores=_NC, num_subcores=_NS)
    body = functools.partial(_sc_aggregate_body, nrows_table, e_rows, half,
                             acc_rows, zrows, chunks_per_tile)
    fn = pl.kernel(
        body,
        out_type=jax.ShapeDtypeStruct((num_dst, _H), jnp.float32),
        mesh=mesh,
        scratch_types=[
            pltpu.VMEM((_CROWS, 128), jnp.int32),
            pltpu.VMEM((_CROWS, 128), jnp.int32),
            pltpu.VMEM((_CHUNK, _H), jnp.float32),
            pltpu.VMEM_SHARED((acc_rows, _H), jnp.float32),
            pltpu.SemaphoreType.DMA,
        ],
        compiler_params=pltpu.CompilerParams(use_tc_tiling_on_sc=False),
    )
    return fn(src2d, dst2d, table, zeros)


_BLK = 1000


def _update_block(h_ref, aggr_ref, w1t_ref, b1_ref, w2at_ref, w2bt_ref, b2_ref, out_ref):
    aggr = aggr_ref[...]
    msg = jnp.maximum(
        jnp.dot(aggr, w1t_ref[...], preferred_element_type=jnp.float32) + b1_ref[...],
        0.0,
    )
    out = (
        jnp.dot(h_ref[...], w2at_ref[...], preferred_element_type=jnp.float32)
        + jnp.dot(msg, w2bt_ref[...], preferred_element_type=jnp.float32)
        + b2_ref[...]
    )
    n = jnp.sqrt(jnp.sum(out * out, axis=1, keepdims=True))
    out_ref[...] = out / jnp.maximum(n, 1e-12)


def _dense_update(h, aggr, W1, b1, W2, b2):
    n, hdim = h.shape
    assert n % _BLK == 0
    grid = (n // _BLK,)
    w1t = W1.T
    w2at = W2[:, :hdim].T
    w2bt = W2[:, hdim:].T
    b1r = b1.reshape(1, hdim)
    b2r = b2.reshape(1, hdim)
    row_spec = pl.BlockSpec((_BLK, hdim), lambda i: (i, 0))
    full_spec = pl.BlockSpec((hdim, hdim), lambda i: (0, 0))
    bias_spec = pl.BlockSpec((1, hdim), lambda i: (0, 0))
    return pl.pallas_call(
        _update_block,
        grid=grid,
        in_specs=[row_spec, row_spec, full_spec, bias_spec, full_spec, full_spec, bias_spec],
        out_specs=row_spec,
        out_shape=jax.ShapeDtypeStruct((n, hdim), jnp.float32),
    )(h, aggr, w1t, b1r, w2at, w2bt, b2r)


def kernel(user_song_adj, song_artist_adj, user_emb, song_emb, artist_emb,
           W_as, b_as, W_s, b_s, W_su, b_su, W_u, b_u):
    num_users = user_emb.shape[0]
    num_songs = song_emb.shape[0]
    aggr_artist = _sc_aggregate(song_artist_adj[0], song_artist_adj[1],
                                artist_emb, num_songs)
    h_s_new = _dense_update(song_emb, aggr_artist, W_as, b_as, W_s, b_s)
    aggr_song = _sc_aggregate(user_song_adj[0], user_song_adj[1],
                              h_s_new, num_users)
    h_u_new = _dense_update(user_emb, aggr_song, W_su, b_su, W_u, b_u)
    return (h_u_new, h_s_new)
